# baseline (speedup 1.0000x reference)
import functools

import jax
import jax.numpy as jnp
from jax import lax
from jax.experimental import pallas as pl
from jax.experimental.pallas import tpu as pltpu
from jax.experimental.pallas import tpu_sc as plsc

_NC = 2
_NS = 16


def _sc_gather(x_hbm, out_hbm, idx_v, out_v, sem):
    wid = lax.axis_index("s") * _NC + lax.axis_index("c")

    @pl.when(wid < 25)
    def _():
        i = wid // 5
        j = wid % 5
        lane = lax.iota(jnp.int32, 16)
        base = j * 32768 + 1 + i
        for t in range(8):
            idx_v[pl.ds(t * 16, 16)] = (lane + t * 16) * 163840 + base
        pltpu.async_copy(x_hbm.at[idx_v], out_v, sem).wait()
        pltpu.sync_copy(out_v, out_hbm.at[pl.ds(wid * 128, 128)])


@functools.partial(
    pl.kernel,
    out_type=jax.ShapeDtypeStruct((3200,), jnp.float32),
    mesh=plsc.VectorSubcoreMesh(core_axis_name="c", subcore_axis_name="s"),
    scratch_types=[
        pltpu.VMEM((128,), jnp.int32),
        pltpu.VMEM((128,), jnp.float32),
        pltpu.SemaphoreType.DMA,
    ],
)
def _sc_kernel(x_hbm, out_hbm, idx_v, out_v, sem):
    _sc_gather(x_hbm, out_hbm, idx_v, out_v, sem)


def kernel(x):
    xin = jnp.transpose(x, (0, 1, 3, 2)).reshape(20971520)
    yflat = _sc_kernel(xin)
    return jnp.transpose(yflat.reshape(5, 5, 128), (2, 0, 1)).reshape(128, 1, 5, 5)


# speedup vs baseline: 1.0216x; 1.0216x over previous
import functools

import jax
import jax.numpy as jnp
from jax import lax
from jax.experimental import pallas as pl
from jax.experimental.pallas import tpu as pltpu
from jax.experimental.pallas import tpu_sc as plsc

_NC = 2
_NS = 16


def _sc_gather(x_hbm, out_hbm, idx_v, out_v, sem):
    wid = lax.axis_index("s")
    lane = lax.iota(jnp.int32, 16)
    for rr in range(2):
        r = wid * 2 + rr

        @pl.when(r < 25)
        def _():
            i = r // 5
            j = r % 5
            base = j * 32768 + 1 + i
            for t in range(8):
                idx_v[pl.ds(t * 16, 16)] = (lane + t * 16) * 163840 + base
            pltpu.async_copy(x_hbm.at[idx_v], out_v, sem).wait()
            pltpu.sync_copy(out_v, out_hbm.at[pl.ds(r * 128, 128)])


@functools.partial(
    pl.kernel,
    out_type=jax.ShapeDtypeStruct((3200,), jnp.float32),
    mesh=plsc.VectorSubcoreMesh(
        core_axis_name="c", subcore_axis_name="s", num_cores=1
    ),
    scratch_types=[
        pltpu.VMEM((128,), jnp.int32),
        pltpu.VMEM((128,), jnp.float32),
        pltpu.SemaphoreType.DMA,
    ],
)
def _sc_kernel(x_hbm, out_hbm, idx_v, out_v, sem):
    _sc_gather(x_hbm, out_hbm, idx_v, out_v, sem)


def kernel(x):
    xin = jnp.transpose(x, (0, 1, 3, 2)).reshape(20971520)
    yflat = _sc_kernel(xin)
    return jnp.transpose(yflat.reshape(5, 5, 128), (2, 0, 1)).reshape(128, 1, 5, 5)


# speedup vs baseline: 1.0234x; 1.0017x over previous
import functools

import jax
import jax.numpy as jnp
from jax import lax
from jax.experimental import pallas as pl
from jax.experimental.pallas import tpu as pltpu
from jax.experimental.pallas import tpu_sc as plsc


def _sc_gather(x_hbm, out_hbm, idx_v, out_v, sem):
    wid = lax.axis_index("s")
    lane = lax.iota(jnp.int32, 16)
    for rr in range(2):
        r = wid * 2 + rr

        @pl.when(r < 25)
        def _():
            i = r // 5
            j = r % 5
            base = j * 32768 + 1 + i
            for t in range(8):
                idx_v[pl.ds(t * 16, 16)] = (lane + t * 16) * 163840 + base
            pltpu.async_copy(x_hbm.at[idx_v], out_v, sem).wait()
            pltpu.sync_copy(out_v, out_hbm.at[pl.ds(r * 128, 128)])


@functools.partial(
    pl.kernel,
    out_type=jax.ShapeDtypeStruct((3200,), jnp.float32),
    mesh=plsc.VectorSubcoreMesh(
        core_axis_name="c", subcore_axis_name="s", num_cores=1
    ),
    scratch_types=[
        pltpu.VMEM((128,), jnp.int32),
        pltpu.VMEM((128,), jnp.float32),
        pltpu.SemaphoreType.DMA,
    ],
)
def _sc_kernel(x_hbm, out_hbm, idx_v, out_v, sem):
    _sc_gather(x_hbm, out_hbm, idx_v, out_v, sem)


def kernel(x):
    xin = jnp.transpose(x, (0, 1, 3, 2)).reshape(20971520)
    yflat = _sc_kernel(xin)
    return jnp.transpose(yflat.reshape(5, 5, 128), (2, 0, 1)).reshape(128, 1, 5, 5)
